# baseline (speedup 1.0000x reference)
import functools

import jax
import jax.numpy as jnp
from jax import lax
from jax.experimental import pallas as pl
from jax.experimental.pallas import tpu as pltpu
from jax.experimental.pallas import tpu_sc as plsc

_LANES = 16


@functools.lru_cache(maxsize=None)
def _build_sc_embed(N, V, D, T, NW, b_per_w, C):
    mesh = plsc.VectorSubcoreMesh(core_axis_name="c", subcore_axis_name="s")

    @functools.partial(
        pl.kernel,
        mesh=mesh,
        out_type=jax.ShapeDtypeStruct((N, D), jnp.float32),
        scratch_types=[
            pltpu.VMEM((b_per_w,), jnp.int32),
            pltpu.VMEM((C, D), jnp.float32),
            pltpu.VMEM((C, D), jnp.float32),
            pltpu.SemaphoreType.DMA,
            pltpu.SemaphoreType.DMA,
        ],
    )
    def sc_embed(ids_hbm, tok_hbm, pos_hbm, out_hbm, idx_v, tok_v, pos_v,
                 sem_g, sem_p):
        wid = lax.axis_index("s") * 2 + lax.axis_index("c")
        base = wid * b_per_w
        t_base = lax.rem(base, T)
        pltpu.sync_copy(ids_hbm.at[pl.ds(base, b_per_w)], idx_v)

        def chunk(ci, _):
            off = ci * C
            g = pltpu.async_copy(
                tok_hbm.at[idx_v.at[pl.ds(off, C)]], tok_v, sem_g)
            p = pltpu.async_copy(
                pos_hbm.at[pl.ds(t_base + off, C)], pos_v, sem_p)
            g.wait()
            p.wait()

            def add_row(r, _):
                def add_vec(j, _):
                    s = pl.ds(j * _LANES, _LANES)
                    tok_v[r, s] = tok_v[r, s] + pos_v[r, s]
                    return 0
                lax.fori_loop(0, D // _LANES, add_vec, 0)
                return 0

            lax.fori_loop(0, C, add_row, 0)
            pltpu.sync_copy(tok_v, out_hbm.at[pl.ds(base + off, C)])
            return 0

        lax.fori_loop(0, b_per_w // C, chunk, 0)

    return sc_embed


def kernel(input_ids, token_table, pos_table):
    B, T = input_ids.shape
    V, D = token_table.shape
    N = B * T
    NW = 32
    b_per_w = N // NW
    C = 32
    flat_ids = input_ids.reshape(N).astype(jnp.int32)
    fn = _build_sc_embed(N, V, D, T, NW, b_per_w, C)
    out = fn(flat_ids, token_table, pos_table)
    return out.reshape(B, T, D)


# speedup vs baseline: 2.4774x; 2.4774x over previous
import functools

import jax
import jax.numpy as jnp
from jax import lax
from jax.experimental import pallas as pl
from jax.experimental.pallas import tpu as pltpu
from jax.experimental.pallas import tpu_sc as plsc

_LANES = 16


@functools.lru_cache(maxsize=None)
def _build_sc_embed(N, V, D, T, NW, b_per_w, C):
    mesh = plsc.VectorSubcoreMesh(core_axis_name="c", subcore_axis_name="s")
    nchunk = b_per_w // C

    @functools.partial(
        pl.kernel,
        mesh=mesh,
        out_type=jax.ShapeDtypeStruct((N, D), jnp.float32),
        scratch_types=[
            pltpu.VMEM((b_per_w,), jnp.int32),
            pltpu.VMEM((C, D), jnp.float32),
            pltpu.VMEM((C, D), jnp.float32),
            pltpu.VMEM((C, D), jnp.float32),
            pltpu.VMEM((C, D), jnp.float32),
            pltpu.SemaphoreType.DMA,
            pltpu.SemaphoreType.DMA,
        ],
    )
    def sc_embed(ids_hbm, tok_hbm, pos_hbm, out_hbm, idx_v,
                 tok0, pos0, tok1, pos1, sem0, sem1):
        wid = lax.axis_index("s") * 2 + lax.axis_index("c")
        base = wid * b_per_w
        t_base = lax.rem(base, T)
        pltpu.sync_copy(ids_hbm.at[pl.ds(base, b_per_w)], idx_v)

        def start(c, tok, pos, sem):
            off = c * C
            pltpu.async_copy(tok_hbm.at[idx_v.at[pl.ds(off, C)]], tok, sem)
            pltpu.async_copy(pos_hbm.at[pl.ds(t_base + off, C)], pos, sem)

        def wait_buf(tok, pos, sem):
            pltpu.make_async_copy(
                tok_hbm.at[idx_v.at[pl.ds(0, C)]], tok, sem).wait()
            pltpu.make_async_copy(pos_hbm.at[pl.ds(0, C)], pos, sem).wait()

        start(0, tok0, pos0, sem0)
        start(1, tok1, pos1, sem1)

        def step(i2, _):
            for b, (tok, pos, sem) in enumerate(
                    ((tok0, pos0, sem0), (tok1, pos1, sem1))):
                c = i2 * 2 + b
                wait_buf(tok, pos, sem)

                def add_row(r, _, tok=tok, pos=pos):
                    for j in range(D // _LANES):
                        sl = pl.ds(j * _LANES, _LANES)
                        plsc.addupdate(tok.at[r, sl], pos[r, sl])
                    return 0

                lax.fori_loop(0, C, add_row, 0)
                pltpu.sync_copy(tok, out_hbm.at[pl.ds(base + c * C, C)])

                nc = c + 2

                @pl.when(nc < nchunk)
                def _(tok=tok, pos=pos, sem=sem, nc=nc):
                    start(nc, tok, pos, sem)
            return 0

        lax.fori_loop(0, nchunk // 2, step, 0)

    return sc_embed


def kernel(input_ids, token_table, pos_table):
    B, T = input_ids.shape
    V, D = token_table.shape
    N = B * T
    NW = 32
    b_per_w = N // NW
    C = 16
    flat_ids = input_ids.reshape(N).astype(jnp.int32)
    fn = _build_sc_embed(N, V, D, T, NW, b_per_w, C)
    out = fn(flat_ids, token_table, pos_table)
    return out.reshape(B, T, D)


# speedup vs baseline: 2.5857x; 1.0437x over previous
import functools

import jax
import jax.numpy as jnp
from jax import lax
from jax.experimental import pallas as pl
from jax.experimental.pallas import tpu as pltpu
from jax.experimental.pallas import tpu_sc as plsc

_LANES = 16


@functools.lru_cache(maxsize=None)
def _build_sc_embed(N, V, D, T, B, NW, C):
    P = T // NW
    NPC = P // C
    mesh = plsc.VectorSubcoreMesh(core_axis_name="c", subcore_axis_name="s")

    @functools.partial(
        pl.kernel,
        mesh=mesh,
        out_type=jax.ShapeDtypeStruct((N, D), jnp.float32),
        scratch_types=[
            pltpu.VMEM((B * P,), jnp.int32),
            pltpu.VMEM((C, D), jnp.float32),
            pltpu.VMEM((C, D), jnp.float32),
            pltpu.VMEM((C, D), jnp.float32),
            pltpu.VMEM((C, D), jnp.float32),
            pltpu.SemaphoreType.DMA,
            pltpu.SemaphoreType.DMA,
            pltpu.SemaphoreType.DMA,
            pltpu.SemaphoreType.DMA,
            pltpu.SemaphoreType.DMA,
            pltpu.SemaphoreType.DMA,
        ],
    )
    def sc_embed(ids_hbm, tok_hbm, pos_hbm, out_hbm, idx_v,
                 tok0, tok1, posb0, posb1,
                 sg0, sg1, st0, st1, sp0, sp1):
        wid = lax.axis_index("s") * 2 + lax.axis_index("c")
        pos_base = wid * P
        toks = (tok0, tok1)
        poss = (posb0, posb1)
        sgs = (sg0, sg1)
        sts = (st0, st1)
        sps = (sp0, sp1)

        for b in range(B):
            pltpu.sync_copy(ids_hbm.at[pl.ds(b * T + pos_base, P)],
                            idx_v.at[pl.ds(b * P, P)])

        def start_gather(b_, pc, slot):
            src = tok_hbm.at[idx_v.at[pl.ds(b_ * P + pc * C, C)]]
            pltpu.async_copy(src, toks[slot], sgs[slot])

        def start_pos(pc, slot):
            pltpu.async_copy(pos_hbm.at[pl.ds(pos_base + pc * C, C)],
                             poss[slot], sps[slot])

        def wait_load(dst, sem):
            pltpu.make_async_copy(pos_hbm.at[pl.ds(0, C)], dst, sem).wait()

        def wait_store(slot):
            pltpu.make_async_copy(toks[slot], out_hbm.at[pl.ds(0, C)],
                                  sts[slot]).wait()

        start_pos(0, 0)
        start_gather(0, 0, 0)

        def pc_pair(i2, _):
            for ppc in (0, 1):
                pc = i2 * 2 + ppc

                @pl.when(pc + 1 < NPC)
                def _(pc=pc, ppc=ppc):
                    start_pos(pc + 1, ppc ^ 1)

                wait_load(poss[ppc], sps[ppc])

                for b in range(B):
                    c = pc * B + b
                    s = b % 2
                    o = (b + 1) % 2

                    if b < B - 1:
                        @pl.when(c >= 1)
                        def _(o=o):
                            wait_store(o)
                        start_gather(b + 1, pc, o)
                    else:
                        @pl.when(pc + 1 < NPC)
                        def _(pc=pc, o=o):
                            wait_store(o)
                            start_gather(0, pc + 1, o)

                    wait_load(toks[s], sgs[s])

                    def add_row(r, _, s=s, ppc=ppc):
                        for j in range(D // _LANES):
                            sl = pl.ds(j * _LANES, _LANES)
                            plsc.addupdate(toks[s].at[r, sl], poss[ppc][r, sl])
                        return 0

                    lax.fori_loop(0, C, add_row, 0)
                    pltpu.async_copy(
                        toks[s],
                        out_hbm.at[pl.ds(b * T + pos_base + pc * C, C)],
                        sts[s])
            return 0

        lax.fori_loop(0, NPC // 2, pc_pair, 0)
        wait_store(0)
        wait_store(1)

    return sc_embed


def kernel(input_ids, token_table, pos_table):
    B, T = input_ids.shape
    V, D = token_table.shape
    N = B * T
    NW = 32
    C = 16
    flat_ids = input_ids.reshape(N).astype(jnp.int32)
    fn = _build_sc_embed(N, V, D, T, B, NW, C)
    out = fn(flat_ids, token_table, pos_table)
    return out.reshape(B, T, D)


# speedup vs baseline: 3.1432x; 1.2156x over previous
import functools

import jax
import jax.numpy as jnp
from jax import lax
from jax.experimental import pallas as pl
from jax.experimental.pallas import tpu as pltpu
from jax.experimental.pallas import tpu_sc as plsc

_LANES = 16


@functools.lru_cache(maxsize=None)
def _build_sc_embed(N, V, D, T, B, NW, C):
    P = T // NW
    NPC = P // C
    mesh = plsc.VectorSubcoreMesh(core_axis_name="c", subcore_axis_name="s")

    tok_scratch = [pltpu.VMEM((C, D), jnp.float32) for _ in range(2 * B)]
    sem_scratch = [pltpu.SemaphoreType.DMA for _ in range(6)]

    @functools.partial(
        pl.kernel,
        mesh=mesh,
        out_type=jax.ShapeDtypeStruct((N, D), jnp.float32),
        scratch_types=[
            pltpu.VMEM((B * P,), jnp.int32),
            pltpu.VMEM((C, D), jnp.float32),
            pltpu.VMEM((C, D), jnp.float32),
        ] + tok_scratch + sem_scratch,
    )
    def sc_embed(ids_hbm, tok_hbm, pos_hbm, out_hbm, idx_v, posb0, posb1,
                 *rest):
        toks = (rest[0:B], rest[B:2 * B])
        sg0, sg1, st0, st1, sp0, sp1 = rest[2 * B:]
        poss = (posb0, posb1)
        sgs = (sg0, sg1)
        sts = (st0, st1)
        sps = (sp0, sp1)

        wid = lax.axis_index("s") * 2 + lax.axis_index("c")
        pos_base = wid * P

        for b in range(B):
            pltpu.sync_copy(ids_hbm.at[pl.ds(b * T + pos_base, P)],
                            idx_v.at[pl.ds(b * P, P)])

        def start_gathers(pc, side):
            for b in range(B):
                src = tok_hbm.at[idx_v.at[pl.ds(b * P + pc * C, C)]]
                pltpu.async_copy(src, toks[side][b], sgs[side])

        def start_pos(pc, side):
            pltpu.async_copy(pos_hbm.at[pl.ds(pos_base + pc * C, C)],
                             poss[side], sps[side])

        def wait_n(dst, sem, n):
            for _ in range(n):
                pltpu.make_async_copy(pos_hbm.at[pl.ds(0, C)], dst, sem).wait()

        def wait_stores(side):
            for b in range(B):
                pltpu.make_async_copy(toks[side][b],
                                      out_hbm.at[pl.ds(0, C)],
                                      sts[side]).wait()

        start_pos(0, 0)
        start_gathers(0, 0)

        def pc_pair(i2, _):
            for side in (0, 1):
                pc = i2 * 2 + side

                @pl.when(jnp.logical_and(pc >= 1, pc + 1 < NPC))
                def _(side=side):
                    wait_stores(side ^ 1)

                @pl.when(pc + 1 < NPC)
                def _(pc=pc, side=side):
                    start_gathers(pc + 1, side ^ 1)
                    start_pos(pc + 1, side ^ 1)

                wait_n(poss[side], sps[side], 1)
                wait_n(toks[side][0], sgs[side], B)

                @plsc.parallel_loop(0, C, 1)
                def _(r, side=side):
                    half = D // _LANES // 2
                    for h in range(2):
                        row = [poss[side][r, pl.ds((h * half + j) * _LANES,
                                                   _LANES)]
                               for j in range(half)]
                        for b in range(B):
                            for j in range(half):
                                sl = pl.ds((h * half + j) * _LANES, _LANES)
                                plsc.addupdate(toks[side][b].at[r, sl],
                                               row[j])

                for b in range(B):
                    pltpu.async_copy(
                        toks[side][b],
                        out_hbm.at[pl.ds(b * T + pos_base + pc * C, C)],
                        sts[side])
            return 0

        lax.fori_loop(0, NPC // 2, pc_pair, 0)
        wait_stores(0)
        wait_stores(1)

    return sc_embed


def kernel(input_ids, token_table, pos_table):
    B, T = input_ids.shape
    V, D = token_table.shape
    N = B * T
    NW = 32
    C = 8
    flat_ids = input_ids.reshape(N).astype(jnp.int32)
    fn = _build_sc_embed(N, V, D, T, B, NW, C)
    out = fn(flat_ids, token_table, pos_table)
    return out.reshape(B, T, D)
